# baseline (speedup 1.0000x reference)
import functools

import jax
import jax.numpy as jnp
from jax import lax
from jax.experimental import pallas as pl
from jax.experimental.pallas import tpu as pltpu
from jax.experimental.pallas import tpu_sc as plsc

N = 10000
D = 128
E = 320000
MAXDEG = 20
NUMG = 128

NC = 2
NS = 16
NW = NC * NS
CH = 128
EPT = 10240
EPAD = NW * EPT
NCH = EPT // CH
DCH = EPAD // NS // CH
DSTG = 80
SSTG = 32
C0 = 160
C1 = (NW * NCH - NS * C0) // NS
NPAD = 10240
RPS = NPAD // NS
WA = 624
WB = N - (NS - 1) * WA

BROW = 400
NB = N // BROW


def _degrees_sc(dst4, zdeg, ones16):
    mesh = plsc.VectorSubcoreMesh(core_axis_name="c", subcore_axis_name="s")

    @functools.partial(
        pl.kernel,
        mesh=mesh,
        out_type=jax.ShapeDtypeStruct((4, N, 16), jnp.float32),
        compiler_params=pltpu.CompilerParams(use_tc_tiling_on_sc=False),
        scratch_types=[
            pltpu.VMEM((DSTG, CH), jnp.int32),
            pltpu.VMEM((CH, 16), jnp.float32),
            pltpu.VMEM_SHARED((NPAD, 16), jnp.float32),
            pltpu.VMEM_SHARED((NPAD, 16), jnp.float32),
            pltpu.SemaphoreType.DMA,
        ],
    )
    def k(dst_hbm, zd_hbm, one_hbm, deg_hbm, dst_v, ones_v, accA, accB, sem):
        cid = lax.axis_index("c")
        sid = lax.axis_index("s")
        pltpu.sync_copy(zd_hbm, accA.at[pl.ds(sid * RPS, RPS)])
        pltpu.sync_copy(zd_hbm, accB.at[pl.ds(sid * RPS, RPS)])
        pltpu.sync_copy(one_hbm, ones_v)
        plsc.subcore_barrier()
        for j, acc in ((0, accA), (1, accB)):
            t = 2 * cid + j
            tbase = t * (EPAD // CH) + sid * DCH

            def fire(c, carry, acc=acc):
                pltpu.async_copy(ones_v, acc.at[dst_v.at[c]], sem, add=True)
                return carry

            def drain(c, carry):
                pltpu.make_async_copy(one_hbm, ones_v, sem).wait()
                return carry

            for ph in range(DCH // DSTG):
                pltpu.sync_copy(
                    dst_hbm.at[pl.ds(tbase + ph * DSTG, DSTG)], dst_v)
                lax.fori_loop(0, DSTG, fire, 0)
                lax.fori_loop(0, DSTG, drain, 0)
        plsc.subcore_barrier()
        for j, acc in ((0, accA), (1, accB)):
            t = 2 * cid + j

            @pl.when(sid < NS - 1)
            def _(acc=acc, t=t):
                pltpu.sync_copy(acc.at[pl.ds(sid * WA, WA)],
                                deg_hbm.at[t].at[pl.ds(sid * WA, WA)])

            @pl.when(sid == NS - 1)
            def _(acc=acc, t=t):
                pltpu.sync_copy(acc.at[pl.ds(sid * WA, WB)],
                                deg_hbm.at[t].at[pl.ds(sid * WA, WB)])

    return k(dst4, zdeg, ones16)


def _segsum_sc(h, src2, dst2, zrows):
    mesh = plsc.VectorSubcoreMesh(core_axis_name="c", subcore_axis_name="s")

    @functools.partial(
        pl.kernel,
        mesh=mesh,
        out_type=jax.ShapeDtypeStruct((2 * N, D), jnp.float32),
        scratch_types=[
            pltpu.VMEM((SSTG, CH), jnp.int32),
            pltpu.VMEM((SSTG, CH), jnp.int32),
            pltpu.VMEM((CH, D), jnp.float32),
            pltpu.VMEM((CH, D), jnp.float32),
            pltpu.VMEM_SHARED((NPAD, D), jnp.float32),
            pltpu.SemaphoreType.DMA,
            pltpu.SemaphoreType.DMA,
        ],
    )
    def k(h_hbm, src_hbm, dst_hbm, zr_hbm,
          out_hbm, src_v, dst_v, rowsA, rowsB, acc_sh, semA, semB):
        cid = lax.axis_index("c")
        sid = lax.axis_index("s")
        pltpu.sync_copy(zr_hbm, acc_sh.at[pl.ds(sid * RPS, RPS)])
        plsc.subcore_barrier()

        def run(rows0, nrows):
            for ph in range(nrows // SSTG):
                rbase = rows0 + ph * SSTG
                pltpu.sync_copy(src_hbm.at[pl.ds(rbase, SSTG)], src_v)
                pltpu.sync_copy(dst_hbm.at[pl.ds(rbase, SSTG)], dst_v)
                pltpu.async_copy(h_hbm.at[src_v.at[0]], rowsA, semA)

                def body(i, carry):
                    c0 = 2 * i
                    pltpu.async_copy(h_hbm.at[src_v.at[c0 + 1]], rowsB, semB)
                    pltpu.make_async_copy(h_hbm.at[pl.ds(0, CH)], rowsA,
                                          semA).wait()
                    pltpu.sync_copy(rowsA, acc_sh.at[dst_v.at[c0]], add=True)

                    @pl.when(i < SSTG // 2 - 1)
                    def _():
                        pltpu.async_copy(h_hbm.at[src_v.at[c0 + 2]], rowsA,
                                         semA)

                    pltpu.make_async_copy(h_hbm.at[pl.ds(0, CH)], rowsB,
                                          semB).wait()
                    pltpu.sync_copy(rowsB, acc_sh.at[dst_v.at[c0 + 1]],
                                    add=True)
                    return carry

                lax.fori_loop(0, SSTG // 2, body, 0)

        @pl.when(cid == 0)
        def _():
            run(sid * C0, C0)

        @pl.when(cid == 1)
        def _():
            run(NS * C0 + sid * C1, C1)

        plsc.subcore_barrier()
        ob = cid * N + sid * WA

        @pl.when(sid < NS - 1)
        def _():
            pltpu.sync_copy(acc_sh.at[pl.ds(sid * WA, WA)],
                            out_hbm.at[pl.ds(ob, WA)])

        @pl.when(sid == NS - 1)
        def _():
            pltpu.sync_copy(acc_sh.at[pl.ds(sid * WA, WB)],
                            out_hbm.at[pl.ds(ob, WB)])

    return k(h, src2, dst2, zrows)


def _inproj(x, We, be):
    def k(x_ref, w_ref, b_ref, o_ref):
        o_ref[...] = lax.dot_general(
            x_ref[...], w_ref[...], (((1,), (1,)), ((), ())),
            preferred_element_type=jnp.float32) + b_ref[...]

    return pl.pallas_call(
        k,
        grid=(NB,),
        in_specs=[
            pl.BlockSpec((BROW, D), lambda i: (i, 0)),
            pl.BlockSpec((D, D), lambda i: (0, 0)),
            pl.BlockSpec((1, D), lambda i: (0, 0)),
        ],
        out_specs=pl.BlockSpec((BROW, D), lambda i: (i, 0)),
        out_shape=jax.ShapeDtypeStruct((N, D), jnp.float32),
    )(x, We, be[None, :])


def _deglinear(part, deg4, li, h, Wcat, bl, relu):
    def k(p0_ref, p1_ref, d_ref, h_ref, w_ref, b_ref, o_ref):
        agg = p0_ref[...] + p1_ref[...]
        degf = d_ref[0][:, 0]
        deg = jnp.clip(degf.astype(jnp.int32), 0, MAXDEG)
        cat = jnp.concatenate([agg, h_ref[...]], axis=1)
        acc = jnp.zeros((BROW, D), jnp.float32)
        for d in range(MAXDEG + 1):
            m = (deg == d).astype(jnp.float32)[:, None]
            acc = acc + jnp.dot(cat * m, w_ref[d],
                                preferred_element_type=jnp.float32)
            acc = acc + m * b_ref[d][None, :]
        if relu:
            acc = jnp.maximum(acc, 0.0)
        o_ref[...] = acc

    return pl.pallas_call(
        k,
        grid=(NB,),
        in_specs=[
            pl.BlockSpec((BROW, D), lambda i: (i, 0)),
            pl.BlockSpec((BROW, D), lambda i: (i + NB, 0)),
            pl.BlockSpec((1, BROW, 16), lambda i: (li, i, 0)),
            pl.BlockSpec((BROW, D), lambda i: (i, 0)),
            pl.BlockSpec((MAXDEG + 1, 2 * D, D), lambda i: (0, 0, 0)),
            pl.BlockSpec((MAXDEG + 1, D), lambda i: (0, 0)),
        ],
        out_specs=pl.BlockSpec((BROW, D), lambda i: (i, 0)),
        out_shape=jax.ShapeDtypeStruct((N, D), jnp.float32),
    )(part, part, deg4, h, Wcat, bl)


def _pool_proj(h, bidx3, mask3, Wo, bo):
    def k(h_ref, bi_ref, m_ref, w_ref, b_ref, o_ref, acc_ref):
        i = pl.program_id(0)

        @pl.when(i == 0)
        def _():
            acc_ref[...] = jnp.zeros((NUMG, D), jnp.float32)

        bi = bi_ref[0, 0, :]
        mk = m_ref[0, 0, :]
        oh = (bi[:, None] == lax.broadcasted_iota(jnp.int32, (BROW, NUMG), 1))
        oh = oh.astype(jnp.float32) * mk[:, None]
        acc_ref[...] += lax.dot_general(
            oh, h_ref[...], (((0,), (0,)), ((), ())),
            preferred_element_type=jnp.float32)

        @pl.when(i == NB - 1)
        def _():
            o_ref[...] = lax.dot_general(
                acc_ref[...], w_ref[...], (((1,), (1,)), ((), ())),
                preferred_element_type=jnp.float32) + b_ref[...]

    return pl.pallas_call(
        k,
        grid=(NB,),
        in_specs=[
            pl.BlockSpec((BROW, D), lambda i: (i, 0)),
            pl.BlockSpec((1, 1, BROW), lambda i: (i, 0, 0)),
            pl.BlockSpec((1, 1, BROW), lambda i: (i, 0, 0)),
            pl.BlockSpec((D, D), lambda i: (0, 0)),
            pl.BlockSpec((1, D), lambda i: (0, 0)),
        ],
        out_specs=pl.BlockSpec((NUMG, D), lambda i: (0, 0)),
        out_shape=jax.ShapeDtypeStruct((NUMG, D), jnp.float32),
        scratch_shapes=[pltpu.VMEM((NUMG, D), jnp.float32)],
    )(h, bidx3, mask3, Wo, bo[None, :])


def kernel(x, edge_index, subgraph_edge_index, node_subnode_index,
           subnode_node_index, ground_node, subgraph_batch_index, batch_idx,
           We, be,
           Wl_g, bl_g, Wr_g,
           Wl_gs, bl_gs, Wr_gs,
           Wl_s, bl_s, Wr_s,
           Wl_sg, bl_sg, Wr_sg,
           Wo, bo):
    pad_src = jnp.zeros((EPAD - E,), jnp.int32)
    pad_dst = jnp.full((EPAD - E,), N, jnp.int32)

    def prep(ei):
        src = jnp.concatenate([ei[0], pad_src]).reshape(NW * NCH, CH)
        dst = jnp.concatenate([ei[1], pad_dst]).reshape(NW * NCH, CH)
        return src, dst

    zrows = jnp.zeros((RPS, D), jnp.float32)
    zdeg = jnp.zeros((RPS, 16), jnp.float32)
    ones16 = jnp.ones((CH, 16), jnp.float32)

    layers = [
        (edge_index, Wl_g, bl_g, Wr_g),
        (node_subnode_index, Wl_gs, bl_gs, Wr_gs),
        (subgraph_edge_index, Wl_s, bl_s, Wr_s),
        (subnode_node_index, Wl_sg, bl_sg, Wr_sg),
    ]
    edges = [prep(ei) for ei, _, _, _ in layers]
    dst4 = jnp.concatenate([d for _, d in edges])

    deg4 = _degrees_sc(dst4, zdeg, ones16)

    h = _inproj(x, We, be)
    for li, (_, Wl, bl, Wr) in enumerate(layers):
        src2, dst2 = edges[li]
        part = _segsum_sc(h, src2, dst2, zrows)
        Wcat = jnp.concatenate(
            [jnp.transpose(Wl, (0, 2, 1)), jnp.transpose(Wr, (0, 2, 1))],
            axis=1)
        h = _deglinear(part, deg4, li, h, Wcat, bl, relu=(li < 3))

    bidx3 = batch_idx.reshape(NB, 1, BROW)
    mask3 = ground_node.astype(jnp.float32).reshape(NB, 1, BROW)
    return _pool_proj(h, bidx3, mask3, Wo, bo)


# speedup vs baseline: 2.4844x; 2.4844x over previous
import functools

import jax
import jax.numpy as jnp
from jax import lax
from jax.experimental import pallas as pl
from jax.experimental.pallas import tpu as pltpu
from jax.experimental.pallas import tpu_sc as plsc

N = 10000
D = 128
E = 320000
MAXDEG = 20
NUMG = 128

NC = 2
NS = 16
NW = NC * NS
CH = 128
EPT = 10240
EPAD = NW * EPT
NCH = EPT // CH
DCH = EPAD // NS // CH
DSTG = 80
SSTG = 32
C0 = 128
C1 = (NW * NCH - NS * C0) // NS
NPAD = 10240
RPS = NPAD // NS
WA = 624
WB = N - (NS - 1) * WA

BROW = 400
NB = N // BROW


def _degrees_sc(dst4, zdeg, ones16):
    mesh = plsc.VectorSubcoreMesh(core_axis_name="c", subcore_axis_name="s")

    @functools.partial(
        pl.kernel,
        mesh=mesh,
        out_type=jax.ShapeDtypeStruct((4, N, 16), jnp.float32),
        compiler_params=pltpu.CompilerParams(use_tc_tiling_on_sc=False),
        scratch_types=[
            pltpu.VMEM((DSTG, CH), jnp.int32),
            pltpu.VMEM((CH, 16), jnp.float32),
            pltpu.VMEM_SHARED((NPAD, 16), jnp.float32),
            pltpu.VMEM_SHARED((NPAD, 16), jnp.float32),
            pltpu.SemaphoreType.DMA,
        ],
    )
    def k(dst_hbm, zd_hbm, one_hbm, deg_hbm, dst_v, ones_v, accA, accB, sem):
        cid = lax.axis_index("c")
        sid = lax.axis_index("s")
        pltpu.sync_copy(zd_hbm, accA.at[pl.ds(sid * RPS, RPS)])
        pltpu.sync_copy(zd_hbm, accB.at[pl.ds(sid * RPS, RPS)])
        pltpu.sync_copy(one_hbm, ones_v)
        plsc.subcore_barrier()
        for j, acc in ((0, accA), (1, accB)):
            t = 2 * cid + j
            tbase = t * (EPAD // CH) + sid * DCH

            def fire(c, carry, acc=acc):
                pltpu.async_copy(ones_v, acc.at[dst_v.at[c]], sem, add=True)
                return carry

            def drain(c, carry):
                pltpu.make_async_copy(one_hbm, ones_v, sem).wait()
                return carry

            for ph in range(DCH // DSTG):
                pltpu.sync_copy(
                    dst_hbm.at[pl.ds(tbase + ph * DSTG, DSTG)], dst_v)
                lax.fori_loop(0, DSTG, fire, 0)
                lax.fori_loop(0, DSTG, drain, 0)
        plsc.subcore_barrier()
        for j, acc in ((0, accA), (1, accB)):
            t = 2 * cid + j

            @pl.when(sid < NS - 1)
            def _(acc=acc, t=t):
                pltpu.sync_copy(acc.at[pl.ds(sid * WA, WA)],
                                deg_hbm.at[t].at[pl.ds(sid * WA, WA)])

            @pl.when(sid == NS - 1)
            def _(acc=acc, t=t):
                pltpu.sync_copy(acc.at[pl.ds(sid * WA, WB)],
                                deg_hbm.at[t].at[pl.ds(sid * WA, WB)])

    return k(dst4, zdeg, ones16)


def _segsum_sc(h, src2, dst2, zrows):
    mesh = plsc.VectorSubcoreMesh(core_axis_name="c", subcore_axis_name="s")

    @functools.partial(
        pl.kernel,
        mesh=mesh,
        out_type=jax.ShapeDtypeStruct((2 * N, D), jnp.float32),
        scratch_types=[
            pltpu.VMEM((SSTG, CH), jnp.int32),
            pltpu.VMEM((SSTG, CH), jnp.int32),
            pltpu.VMEM((CH, D), jnp.float32),
            pltpu.VMEM((CH, D), jnp.float32),
            pltpu.VMEM_SHARED((NPAD, D), jnp.float32),
            pltpu.SemaphoreType.DMA,
            pltpu.SemaphoreType.DMA,
        ],
    )
    def k(h_hbm, src_hbm, dst_hbm, zr_hbm,
          out_hbm, src_v, dst_v, rowsA, rowsB, acc_sh, semA, semB):
        cid = lax.axis_index("c")
        sid = lax.axis_index("s")
        pltpu.sync_copy(zr_hbm, acc_sh.at[pl.ds(sid * RPS, RPS)])
        plsc.subcore_barrier()

        def run(rows0, nrows):
            for ph in range(nrows // SSTG):
                rbase = rows0 + ph * SSTG
                pltpu.sync_copy(src_hbm.at[pl.ds(rbase, SSTG)], src_v)
                pltpu.sync_copy(dst_hbm.at[pl.ds(rbase, SSTG)], dst_v)
                pltpu.async_copy(h_hbm.at[src_v.at[0]], rowsA, semA)

                def body(i, carry):
                    c0 = 2 * i
                    pltpu.async_copy(h_hbm.at[src_v.at[c0 + 1]], rowsB, semB)
                    pltpu.make_async_copy(h_hbm.at[pl.ds(0, CH)], rowsA,
                                          semA).wait()
                    pltpu.sync_copy(rowsA, acc_sh.at[dst_v.at[c0]], add=True)

                    @pl.when(i < SSTG // 2 - 1)
                    def _():
                        pltpu.async_copy(h_hbm.at[src_v.at[c0 + 2]], rowsA,
                                         semA)

                    pltpu.make_async_copy(h_hbm.at[pl.ds(0, CH)], rowsB,
                                          semB).wait()
                    pltpu.sync_copy(rowsB, acc_sh.at[dst_v.at[c0 + 1]],
                                    add=True)
                    return carry

                lax.fori_loop(0, SSTG // 2, body, 0)

        @pl.when(cid == 0)
        def _():
            run(sid * C0, C0)

        @pl.when(cid == 1)
        def _():
            run(NS * C0 + sid * C1, C1)

        plsc.subcore_barrier()
        ob = cid * N + sid * WA

        @pl.when(sid < NS - 1)
        def _():
            pltpu.sync_copy(acc_sh.at[pl.ds(sid * WA, WA)],
                            out_hbm.at[pl.ds(ob, WA)])

        @pl.when(sid == NS - 1)
        def _():
            pltpu.sync_copy(acc_sh.at[pl.ds(sid * WA, WB)],
                            out_hbm.at[pl.ds(ob, WB)])

    return k(h, src2, dst2, zrows)


def _inproj(x, We, be):
    def k(x_ref, w_ref, b_ref, o_ref):
        o_ref[...] = lax.dot_general(
            x_ref[...], w_ref[...], (((1,), (1,)), ((), ())),
            preferred_element_type=jnp.float32) + b_ref[...]

    return pl.pallas_call(
        k,
        grid=(NB,),
        in_specs=[
            pl.BlockSpec((BROW, D), lambda i: (i, 0)),
            pl.BlockSpec((D, D), lambda i: (0, 0)),
            pl.BlockSpec((1, D), lambda i: (0, 0)),
        ],
        out_specs=pl.BlockSpec((BROW, D), lambda i: (i, 0)),
        out_shape=jax.ShapeDtypeStruct((N, D), jnp.float32),
    )(x, We, be[None, :])


def _deglinear(part, deg4, li, h, Wcat, bl, relu):
    def k(p0_ref, p1_ref, d_ref, h_ref, w_ref, b_ref, o_ref):
        agg = p0_ref[...] + p1_ref[...]
        degf = d_ref[0][:, 0]
        deg = jnp.clip(degf.astype(jnp.int32), 0, MAXDEG)
        cat = jnp.concatenate([agg, h_ref[...]], axis=1)
        acc = jnp.zeros((BROW, D), jnp.float32)
        for d in range(MAXDEG + 1):
            m = (deg == d).astype(jnp.float32)[:, None]
            acc = acc + jnp.dot(cat * m, w_ref[d],
                                preferred_element_type=jnp.float32)
            acc = acc + m * b_ref[d][None, :]
        if relu:
            acc = jnp.maximum(acc, 0.0)
        o_ref[...] = acc

    return pl.pallas_call(
        k,
        grid=(NB,),
        in_specs=[
            pl.BlockSpec((BROW, D), lambda i: (i, 0)),
            pl.BlockSpec((BROW, D), lambda i: (i + NB, 0)),
            pl.BlockSpec((1, BROW, 16), lambda i: (li, i, 0)),
            pl.BlockSpec((BROW, D), lambda i: (i, 0)),
            pl.BlockSpec((MAXDEG + 1, 2 * D, D), lambda i: (0, 0, 0)),
            pl.BlockSpec((MAXDEG + 1, D), lambda i: (0, 0)),
        ],
        out_specs=pl.BlockSpec((BROW, D), lambda i: (i, 0)),
        out_shape=jax.ShapeDtypeStruct((N, D), jnp.float32),
    )(part, part, deg4, h, Wcat, bl)


def _pool_proj(h, bidx3, mask3, Wo, bo):
    def k(h_ref, bi_ref, m_ref, w_ref, b_ref, o_ref, acc_ref):
        i = pl.program_id(0)

        @pl.when(i == 0)
        def _():
            acc_ref[...] = jnp.zeros((NUMG, D), jnp.float32)

        bi = bi_ref[0, 0, :]
        mk = m_ref[0, 0, :]
        oh = (bi[:, None] == lax.broadcasted_iota(jnp.int32, (BROW, NUMG), 1))
        oh = oh.astype(jnp.float32) * mk[:, None]
        acc_ref[...] += lax.dot_general(
            oh, h_ref[...], (((0,), (0,)), ((), ())),
            preferred_element_type=jnp.float32)

        @pl.when(i == NB - 1)
        def _():
            o_ref[...] = lax.dot_general(
                acc_ref[...], w_ref[...], (((1,), (1,)), ((), ())),
                preferred_element_type=jnp.float32) + b_ref[...]

    return pl.pallas_call(
        k,
        grid=(NB,),
        in_specs=[
            pl.BlockSpec((BROW, D), lambda i: (i, 0)),
            pl.BlockSpec((1, 1, BROW), lambda i: (i, 0, 0)),
            pl.BlockSpec((1, 1, BROW), lambda i: (i, 0, 0)),
            pl.BlockSpec((D, D), lambda i: (0, 0)),
            pl.BlockSpec((1, D), lambda i: (0, 0)),
        ],
        out_specs=pl.BlockSpec((NUMG, D), lambda i: (0, 0)),
        out_shape=jax.ShapeDtypeStruct((NUMG, D), jnp.float32),
        scratch_shapes=[pltpu.VMEM((NUMG, D), jnp.float32)],
    )(h, bidx3, mask3, Wo, bo[None, :])


def kernel(x, edge_index, subgraph_edge_index, node_subnode_index,
           subnode_node_index, ground_node, subgraph_batch_index, batch_idx,
           We, be,
           Wl_g, bl_g, Wr_g,
           Wl_gs, bl_gs, Wr_gs,
           Wl_s, bl_s, Wr_s,
           Wl_sg, bl_sg, Wr_sg,
           Wo, bo):
    iar = jnp.arange(EPAD - E, dtype=jnp.int32)
    pad_src = iar * 997 % N
    pad_dst = N + iar % (NPAD - N)

    def prep(ei):
        src = jnp.concatenate([ei[0], pad_src]).reshape(NW * NCH, CH)
        dst = jnp.concatenate([ei[1], pad_dst]).reshape(NW * NCH, CH)
        return src, dst

    zrows = jnp.zeros((RPS, D), jnp.float32)
    zdeg = jnp.zeros((RPS, 16), jnp.float32)
    ones16 = jnp.ones((CH, 16), jnp.float32)

    layers = [
        (edge_index, Wl_g, bl_g, Wr_g),
        (node_subnode_index, Wl_gs, bl_gs, Wr_gs),
        (subgraph_edge_index, Wl_s, bl_s, Wr_s),
        (subnode_node_index, Wl_sg, bl_sg, Wr_sg),
    ]
    edges = [prep(ei) for ei, _, _, _ in layers]
    dst4 = jnp.concatenate([d for _, d in edges])

    deg4 = _degrees_sc(dst4, zdeg, ones16)

    h = _inproj(x, We, be)
    for li, (_, Wl, bl, Wr) in enumerate(layers):
        src2, dst2 = edges[li]
        part = _segsum_sc(h, src2, dst2, zrows)
        Wcat = jnp.concatenate(
            [jnp.transpose(Wl, (0, 2, 1)), jnp.transpose(Wr, (0, 2, 1))],
            axis=1)
        h = _deglinear(part, deg4, li, h, Wcat, bl, relu=(li < 3))

    bidx3 = batch_idx.reshape(NB, 1, BROW)
    mask3 = ground_node.astype(jnp.float32).reshape(NB, 1, BROW)
    return _pool_proj(h, bidx3, mask3, Wo, bo)


# speedup vs baseline: 3.5294x; 1.4206x over previous
import functools

import jax
import jax.numpy as jnp
from jax import lax
from jax.experimental import pallas as pl
from jax.experimental.pallas import tpu as pltpu
from jax.experimental.pallas import tpu_sc as plsc

N = 10000
D = 128
E = 320000
MAXDEG = 20
NUMG = 128

NC = 2
NS = 16
NW = NC * NS
CH = 128
EPT = 10240
EPAD = NW * EPT
NCH = EPT // CH
DCH = EPAD // NS // CH
DSTG = 80
SSTG = 32
C0 = 80
C1 = (NW * NCH - NS * C0) // NS
NPAD = 10240
RPS = NPAD // NS
WA = 624
WB = N - (NS - 1) * WA

BROW = 400
NB = N // BROW


def _degrees_sc(dst4, zdeg, ones16):
    mesh = plsc.VectorSubcoreMesh(core_axis_name="c", subcore_axis_name="s")

    @functools.partial(
        pl.kernel,
        mesh=mesh,
        out_type=jax.ShapeDtypeStruct((4, N, 16), jnp.float32),
        compiler_params=pltpu.CompilerParams(use_tc_tiling_on_sc=False),
        scratch_types=[
            pltpu.VMEM((DSTG, CH), jnp.int32),
            pltpu.VMEM((CH, 16), jnp.float32),
            pltpu.VMEM_SHARED((NPAD, 16), jnp.float32),
            pltpu.VMEM_SHARED((NPAD, 16), jnp.float32),
            pltpu.SemaphoreType.DMA,
        ],
    )
    def k(dst_hbm, zd_hbm, one_hbm, deg_hbm, dst_v, ones_v, accA, accB, sem):
        cid = lax.axis_index("c")
        sid = lax.axis_index("s")
        pltpu.sync_copy(zd_hbm, accA.at[pl.ds(sid * RPS, RPS)])
        pltpu.sync_copy(zd_hbm, accB.at[pl.ds(sid * RPS, RPS)])
        pltpu.sync_copy(one_hbm, ones_v)
        plsc.subcore_barrier()
        for j, acc in ((0, accA), (1, accB)):
            t = 2 * cid + j
            tbase = t * (EPAD // CH) + sid * DCH

            def fire(c, carry, acc=acc):
                pltpu.async_copy(ones_v, acc.at[dst_v.at[c]], sem, add=True)
                return carry

            def drain(c, carry):
                pltpu.make_async_copy(one_hbm, ones_v, sem).wait()
                return carry

            for ph in range(DCH // DSTG):
                pltpu.sync_copy(
                    dst_hbm.at[pl.ds(tbase + ph * DSTG, DSTG)], dst_v)
                lax.fori_loop(0, DSTG, fire, 0)
                lax.fori_loop(0, DSTG, drain, 0)
        plsc.subcore_barrier()
        for j, acc in ((0, accA), (1, accB)):
            t = 2 * cid + j

            @pl.when(sid < NS - 1)
            def _(acc=acc, t=t):
                pltpu.sync_copy(acc.at[pl.ds(sid * WA, WA)],
                                deg_hbm.at[t].at[pl.ds(sid * WA, WA)])

            @pl.when(sid == NS - 1)
            def _(acc=acc, t=t):
                pltpu.sync_copy(acc.at[pl.ds(sid * WA, WB)],
                                deg_hbm.at[t].at[pl.ds(sid * WA, WB)])

    return k(dst4, zdeg, ones16)


def _segsum_sc(h, src2, dst2, zrows):
    mesh = plsc.VectorSubcoreMesh(core_axis_name="c", subcore_axis_name="s")

    @functools.partial(
        pl.kernel,
        mesh=mesh,
        out_type=jax.ShapeDtypeStruct((2 * N, D), jnp.float32),
        scratch_types=[
            pltpu.VMEM((SSTG, CH), jnp.int32),
            pltpu.VMEM((SSTG, CH), jnp.int32),
            pltpu.VMEM((CH, D), jnp.float32),
            pltpu.VMEM((CH, D), jnp.float32),
            pltpu.VMEM_SHARED((NPAD, D), jnp.float32),
            pltpu.SemaphoreType.DMA,
            pltpu.SemaphoreType.DMA,
        ],
    )
    def k(h_hbm, src_hbm, dst_hbm, zr_hbm,
          out_hbm, src_v, dst_v, rowsA, rowsB, acc_sh, semA, semB):
        cid = lax.axis_index("c")
        sid = lax.axis_index("s")
        pltpu.sync_copy(zr_hbm, acc_sh.at[pl.ds(sid * RPS, RPS)])
        plsc.subcore_barrier()

        def run(rows0, nrows):
            for ph in range(nrows // SSTG):
                rbase = rows0 + ph * SSTG
                pltpu.sync_copy(src_hbm.at[pl.ds(rbase, SSTG)], src_v)
                pltpu.sync_copy(dst_hbm.at[pl.ds(rbase, SSTG)], dst_v)
                pltpu.async_copy(h_hbm.at[src_v.at[0]], rowsA, semA)

                def body(i, carry):
                    c0 = 2 * i
                    pltpu.async_copy(h_hbm.at[src_v.at[c0 + 1]], rowsB, semB)
                    pltpu.make_async_copy(h_hbm.at[pl.ds(0, CH)], rowsA,
                                          semA).wait()
                    pltpu.sync_copy(rowsA, acc_sh.at[dst_v.at[c0]], add=True)

                    @pl.when(i < SSTG // 2 - 1)
                    def _():
                        pltpu.async_copy(h_hbm.at[src_v.at[c0 + 2]], rowsA,
                                         semA)

                    pltpu.make_async_copy(h_hbm.at[pl.ds(0, CH)], rowsB,
                                          semB).wait()
                    pltpu.sync_copy(rowsB, acc_sh.at[dst_v.at[c0 + 1]],
                                    add=True)
                    return carry

                lax.fori_loop(0, SSTG // 2, body, 0)

        @pl.when(cid == 0)
        def _():
            run(sid * C0, C0)

        @pl.when(cid == 1)
        def _():
            run(NS * C0 + sid * C1, C1)

        plsc.subcore_barrier()
        ob = cid * N + sid * WA

        @pl.when(sid < NS - 1)
        def _():
            pltpu.sync_copy(acc_sh.at[pl.ds(sid * WA, WA)],
                            out_hbm.at[pl.ds(ob, WA)])

        @pl.when(sid == NS - 1)
        def _():
            pltpu.sync_copy(acc_sh.at[pl.ds(sid * WA, WB)],
                            out_hbm.at[pl.ds(ob, WB)])

    return k(h, src2, dst2, zrows)


def _inproj(x, We, be):
    def k(x_ref, w_ref, b_ref, o_ref):
        o_ref[...] = lax.dot_general(
            x_ref[...], w_ref[...], (((1,), (1,)), ((), ())),
            preferred_element_type=jnp.float32) + b_ref[...]

    return pl.pallas_call(
        k,
        grid=(NB,),
        in_specs=[
            pl.BlockSpec((BROW, D), lambda i: (i, 0)),
            pl.BlockSpec((D, D), lambda i: (0, 0)),
            pl.BlockSpec((1, D), lambda i: (0, 0)),
        ],
        out_specs=pl.BlockSpec((BROW, D), lambda i: (i, 0)),
        out_shape=jax.ShapeDtypeStruct((N, D), jnp.float32),
    )(x, We, be[None, :])


def _deglinear(part, deg4, li, h, Wcat, bl, relu):
    def k(p0_ref, p1_ref, d_ref, h_ref, w_ref, b_ref, o_ref):
        agg = p0_ref[...] + p1_ref[...]
        degf = d_ref[0][:, 0]
        deg = jnp.clip(degf.astype(jnp.int32), 0, MAXDEG)
        cat = jnp.concatenate([agg, h_ref[...]], axis=1)
        acc = jnp.zeros((BROW, D), jnp.float32)
        for d in range(MAXDEG + 1):
            m = (deg == d).astype(jnp.float32)[:, None]
            acc = acc + jnp.dot(cat * m, w_ref[d],
                                preferred_element_type=jnp.float32)
            acc = acc + m * b_ref[d][None, :]
        if relu:
            acc = jnp.maximum(acc, 0.0)
        o_ref[...] = acc

    return pl.pallas_call(
        k,
        grid=(NB,),
        in_specs=[
            pl.BlockSpec((BROW, D), lambda i: (i, 0)),
            pl.BlockSpec((BROW, D), lambda i: (i + NB, 0)),
            pl.BlockSpec((1, BROW, 16), lambda i: (li, i, 0)),
            pl.BlockSpec((BROW, D), lambda i: (i, 0)),
            pl.BlockSpec((MAXDEG + 1, 2 * D, D), lambda i: (0, 0, 0)),
            pl.BlockSpec((MAXDEG + 1, D), lambda i: (0, 0)),
        ],
        out_specs=pl.BlockSpec((BROW, D), lambda i: (i, 0)),
        out_shape=jax.ShapeDtypeStruct((N, D), jnp.float32),
    )(part, part, deg4, h, Wcat, bl)


def _pool_proj(h, bidx3, mask3, Wo, bo):
    def k(h_ref, bi_ref, m_ref, w_ref, b_ref, o_ref, acc_ref):
        i = pl.program_id(0)

        @pl.when(i == 0)
        def _():
            acc_ref[...] = jnp.zeros((NUMG, D), jnp.float32)

        bi = bi_ref[0, 0, :]
        mk = m_ref[0, 0, :]
        oh = (bi[:, None] == lax.broadcasted_iota(jnp.int32, (BROW, NUMG), 1))
        oh = oh.astype(jnp.float32) * mk[:, None]
        acc_ref[...] += lax.dot_general(
            oh, h_ref[...], (((0,), (0,)), ((), ())),
            preferred_element_type=jnp.float32)

        @pl.when(i == NB - 1)
        def _():
            o_ref[...] = lax.dot_general(
                acc_ref[...], w_ref[...], (((1,), (1,)), ((), ())),
                preferred_element_type=jnp.float32) + b_ref[...]

    return pl.pallas_call(
        k,
        grid=(NB,),
        in_specs=[
            pl.BlockSpec((BROW, D), lambda i: (i, 0)),
            pl.BlockSpec((1, 1, BROW), lambda i: (i, 0, 0)),
            pl.BlockSpec((1, 1, BROW), lambda i: (i, 0, 0)),
            pl.BlockSpec((D, D), lambda i: (0, 0)),
            pl.BlockSpec((1, D), lambda i: (0, 0)),
        ],
        out_specs=pl.BlockSpec((NUMG, D), lambda i: (0, 0)),
        out_shape=jax.ShapeDtypeStruct((NUMG, D), jnp.float32),
        scratch_shapes=[pltpu.VMEM((NUMG, D), jnp.float32)],
    )(h, bidx3, mask3, Wo, bo[None, :])


def kernel(x, edge_index, subgraph_edge_index, node_subnode_index,
           subnode_node_index, ground_node, subgraph_batch_index, batch_idx,
           We, be,
           Wl_g, bl_g, Wr_g,
           Wl_gs, bl_gs, Wr_gs,
           Wl_s, bl_s, Wr_s,
           Wl_sg, bl_sg, Wr_sg,
           Wo, bo):
    iar = jnp.arange(EPAD - E, dtype=jnp.int32)
    pad_src = iar * 997 % N
    pad_dst = N + iar % (NPAD - N)

    def prep(ei):
        src = jnp.concatenate([ei[0], pad_src]).reshape(NW * NCH, CH)
        dst = jnp.concatenate([ei[1], pad_dst]).reshape(NW * NCH, CH)
        return src, dst

    zrows = jnp.zeros((RPS, D), jnp.float32)
    zdeg = jnp.zeros((RPS, 16), jnp.float32)
    ones16 = jnp.ones((CH, 16), jnp.float32)

    layers = [
        (edge_index, Wl_g, bl_g, Wr_g),
        (node_subnode_index, Wl_gs, bl_gs, Wr_gs),
        (subgraph_edge_index, Wl_s, bl_s, Wr_s),
        (subnode_node_index, Wl_sg, bl_sg, Wr_sg),
    ]
    edges = [prep(ei) for ei, _, _, _ in layers]
    dst4 = jnp.concatenate([d for _, d in edges])

    deg4 = _degrees_sc(dst4, zdeg, ones16)

    h = _inproj(x, We, be)
    for li, (_, Wl, bl, Wr) in enumerate(layers):
        src2, dst2 = edges[li]
        part = _segsum_sc(h, src2, dst2, zrows)
        Wcat = jnp.concatenate(
            [jnp.transpose(Wl, (0, 2, 1)), jnp.transpose(Wr, (0, 2, 1))],
            axis=1)
        h = _deglinear(part, deg4, li, h, Wcat, bl, relu=(li < 3))

    bidx3 = batch_idx.reshape(NB, 1, BROW)
    mask3 = ground_node.astype(jnp.float32).reshape(NB, 1, BROW)
    return _pool_proj(h, bidx3, mask3, Wo, bo)
